# Initial kernel scaffold; baseline (speedup 1.0000x reference)
import jax
import jax.numpy as jnp
from jax.experimental import pallas as pl


def kernel(s_x, q_x, params, s_edge_index, q_edge_index, s_batch, q_batch, s_label):
    raise NotImplementedError("write your pallas kernel here")


# speedup vs baseline: 1.4165x; 1.4165x over previous
import functools

import jax
import jax.numpy as jnp
from jax import lax
from jax.experimental import pallas as pl
from jax.experimental.pallas import tpu as pltpu
from jax.experimental.pallas import tpu_sc as plsc

N = 10000
E = 160000
G = 64

SC_NC = 2
SC_NS = 16
NW = SC_NC * SC_NS
NPASS = 2
NB = NPASS * NW
BR = 160
RB = 64
SELCAP_ROWS = 2560
CHUNK = 3200
NCHUNKS = E // CHUNK
STG_ROWS = 56
FLUSH_ROWS = 48
FLUSH_N = FLUSH_ROWS * RB

RT = 400
NBLK = N // RT


@functools.cache
def _sc_mesh():
    return plsc.VectorSubcoreMesh(core_axis_name="c", subcore_axis_name="s",
                                  num_cores=SC_NC, num_subcores=SC_NS)


def _compact_body(src_hbm, dst_hbm, sel_hbm, counts_hbm,
                  src_v, dst_v, stg0_v, stg1_v, cnt_v):
    c = lax.axis_index("c")
    t = lax.axis_index("s")
    w = c * SC_NS + t
    b0 = w
    b1 = NW + w
    lo0 = b0 * BR
    lo1 = b1 * BR

    def flush(stg_v, bkt, cnt, wr):
        def do(args):
            cnt_, wr_ = args
            wr_a = pl.multiple_of(wr_, 8)
            pltpu.sync_copy(stg_v.at[pl.ds(0, FLUSH_ROWS)],
                            sel_hbm.at[bkt, pl.ds(wr_a, FLUSH_ROWS)])
            for k in range(RB // 16):
                stg_v[0, pl.ds(k * 16, 16)] = stg_v[FLUSH_ROWS,
                                                    pl.ds(k * 16, 16)]
            return cnt_ - FLUSH_N, wr_ + FLUSH_ROWS
        return lax.cond(cnt >= FLUSH_N, do, lambda a: a, (cnt, wr))

    def chunk_loop(ch, carry):
        cnt0, wr0, cnt1, wr1 = carry
        pltpu.sync_copy(src_hbm.at[pl.ds(ch * CHUNK, CHUNK)], src_v)
        pltpu.sync_copy(dst_hbm.at[pl.ds(ch * CHUNK, CHUNK)], dst_v)

        def step(i, carry):
            cnt0, wr0, cnt1, wr1 = carry
            s = src_v[pl.ds(i * 16, 16)]
            d = dst_v[pl.ds(i * 16, 16)]

            def bucket(stg_v, lo, cnt):
                m = (d >= lo) & (d < lo + BR)
                mi = m.astype(jnp.int32)
                val = lax.shift_left(s, 8) | (d - lo)
                offs = cnt + plsc.cumsum(mi) - mi
                plsc.store_scatter(stg_v, [offs // RB, offs % RB], val,
                                   mask=m)
                return cnt + jnp.sum(mi)

            cnt0 = bucket(stg0_v, lo0, cnt0)
            cnt1 = bucket(stg1_v, lo1, cnt1)
            cnt0, wr0 = flush(stg0_v, b0, cnt0, wr0)
            cnt1, wr1 = flush(stg1_v, b1, cnt1, wr1)
            return cnt0, wr0, cnt1, wr1

        return lax.fori_loop(0, CHUNK // 16, step,
                             (cnt0, wr0, cnt1, wr1))

    z = jnp.int32(0)
    cnt0, wr0, cnt1, wr1 = lax.fori_loop(0, NCHUNKS, chunk_loop,
                                         (z, z, z, z))

    iota = lax.iota(jnp.int32, 16)
    trash = jnp.full((16,), BR, jnp.int32)

    def finish(stg_v, bkt, cnt, wr):
        for k in range(RB // 16):
            offs = cnt + k * 16 + iota
            plsc.store_scatter(stg_v, [offs // RB, offs % RB], trash)
        pltpu.sync_copy(stg_v,
                        sel_hbm.at[bkt, pl.ds(pl.multiple_of(wr, 8),
                                              STG_ROWS)])
        total = wr * RB + cnt
        cnt_v[...] = jnp.full((16,), total, jnp.int32)
        pltpu.sync_copy(cnt_v, counts_hbm.at[bkt])

    finish(stg0_v, b0, cnt0, wr0)
    finish(stg1_v, b1, cnt1, wr1)


@functools.cache
def _compact_kernel():
    return pl.kernel(
        _compact_body,
        out_type=(
            jax.ShapeDtypeStruct((NB, SELCAP_ROWS, RB), jnp.int32),
            jax.ShapeDtypeStruct((NB, 16), jnp.int32),
        ),
        mesh=_sc_mesh(),
        scratch_types=[
            pltpu.VMEM((CHUNK,), jnp.int32),
            pltpu.VMEM((CHUNK,), jnp.int32),
            pltpu.VMEM((STG_ROWS, RB), jnp.int32),
            pltpu.VMEM((STG_ROWS, RB), jnp.int32),
            pltpu.VMEM((16,), jnp.int32),
        ],
        compiler_params=pltpu.CompilerParams(needs_layout_passes=False),
    )


def _compact_call(edge_index):
    return _compact_kernel()(edge_index[0], edge_index[1])


def _agg_body(Hf, x_hbm, sel_hbm, counts_hbm, agg_hbm,
              sel_v, idx_v, ld_v, cnt_v, rows_v, acc_v, sem):
    c = lax.axis_index("c")
    t = lax.axis_index("s")
    w = c * SC_NS + t
    colv = lax.iota(jnp.int32, 16)
    vpr = Hf // 16
    zeros = jnp.zeros((16,), jnp.float32)

    for p in range(NPASS):
        b = p * NW + w

        def zf(i, _):
            acc_v[i // vpr, pl.ds((i % vpr) * 16, 16)] = zeros
            return 0
        lax.fori_loop(0, BR * vpr, zf, 0)

        pltpu.sync_copy(counts_hbm.at[b], cnt_v)
        m_tot = cnt_v[...][0]
        nblk = (m_tot + RB - 1) // RB
        nsb = (nblk + 15) // 16

        def sb_loop(sb, _):
            pltpu.sync_copy(
                sel_hbm.at[b, pl.ds(pl.multiple_of(sb * 16, 16), 16)], sel_v)
            hi = jnp.minimum(nblk - sb * 16, 16)

            def blk_loop(r, _):
                for k in range(RB // 16):
                    v = sel_v[r, pl.ds(k * 16, 16)]
                    idx_v[pl.ds(k * 16, 16)] = lax.shift_right_logical(v, 8)
                    ld_v[pl.ds(k * 16, 16)] = v & 255
                pltpu.async_copy(x_hbm.at[idx_v], rows_v, sem).wait()
                for g in range(RB // 16):
                    ldv = ld_v[pl.ds(g * 16, 16)]

                    def edge_loop(l, _):
                        spl = jnp.take(ldv, jnp.full((16,), l, jnp.int32))
                        for j in range(vpr):
                            val = rows_v[g * 16 + l, pl.ds(j * 16, 16)]
                            plsc.addupdate_scatter(acc_v,
                                                   [spl, j * 16 + colv], val)
                        return 0
                    lax.fori_loop(0, 16, edge_loop, 0)
                return 0
            lax.fori_loop(0, hi, blk_loop, 0)
            return 0
        lax.fori_loop(0, nsb, sb_loop, 0)

        base = pl.multiple_of(b * BR, 32)
        if p == 0:
            pltpu.sync_copy(acc_v.at[pl.ds(0, BR)],
                            agg_hbm.at[pl.ds(base, BR)])
        else:
            @pl.when(w < NW - 2)
            def _():
                pltpu.sync_copy(acc_v.at[pl.ds(0, BR)],
                                agg_hbm.at[pl.ds(base, BR)])

            @pl.when(w == NW - 2)
            def _():
                pltpu.sync_copy(acc_v.at[pl.ds(0, 80)],
                                agg_hbm.at[pl.ds(base, 80)])


@functools.cache
def _make_agg_call(Hf):
    return pl.kernel(
        functools.partial(_agg_body, Hf),
        out_type=jax.ShapeDtypeStruct((N, Hf), jnp.float32),
        mesh=_sc_mesh(),
        scratch_types=[
            pltpu.VMEM((16, RB), jnp.int32),
            pltpu.VMEM((RB,), jnp.int32),
            pltpu.VMEM((RB,), jnp.int32),
            pltpu.VMEM((16,), jnp.int32),
            pltpu.VMEM((RB, Hf), jnp.float32),
            pltpu.VMEM((BR + 8, Hf), jnp.float32),
            pltpu.SemaphoreType.DMA,
        ],
        compiler_params=pltpu.CompilerParams(needs_layout_passes=False),
    )


def _segment_sum(x, sel, counts):
    return _make_agg_call(x.shape[1])(x, sel, counts)


def _mlp1_body(eps_ref, x_ref, agg_ref, w1_ref, b1_ref, h1_ref):
    h_in = (1.0 + eps_ref[0, 0]) * x_ref[...] + agg_ref[...]
    h1 = jnp.dot(h_in, w1_ref[...], preferred_element_type=jnp.float32)
    h1_ref[...] = h1 + b1_ref[...]


def _mlp2_body(a_ref, w2_ref, b2_ref, h2_ref):
    h2 = jnp.dot(a_ref[...], w2_ref[...], preferred_element_type=jnp.float32)
    h2_ref[...] = h2 + b2_ref[...]


def _pool_body(b3_ref, x_ref, out_ref):
    i = pl.program_id(0)

    @pl.when(i == 0)
    def _():
        out_ref[...] = jnp.zeros_like(out_ref)

    gids = lax.broadcasted_iota(jnp.int32, (G, 1), 0)
    onehot = (b3_ref[0] == gids).astype(jnp.float32)
    out_ref[...] += jnp.dot(onehot, x_ref[...],
                            preferred_element_type=jnp.float32,
                            precision=lax.Precision.HIGHEST)


def _head_body(p_ref, w1_ref, b1_ref, g_ref, be_ref, w2_ref, b2_ref, out_ref):
    h = jnp.dot(p_ref[...], w1_ref[...], preferred_element_type=jnp.float32)
    h = h + b1_ref[...]
    mean = jnp.mean(h, axis=0, keepdims=True)
    var = jnp.mean(h * h, axis=0, keepdims=True) - mean * mean
    h = (h - mean) * lax.rsqrt(var + 1e-5) * g_ref[...] + be_ref[...]
    h = jnp.maximum(h, 0.0)
    out_ref[...] = jnp.dot(h, w2_ref[...],
                           preferred_element_type=jnp.float32) + b2_ref[...]


def _row_spec(bs):
    return pl.BlockSpec(bs, lambda i: (i, 0))


def _fix_spec(bs):
    return pl.BlockSpec(bs, lambda i: (0, 0))


def _mlp1_call(eps, x, agg, w1, b1):
    in_c = x.shape[1]
    ch = w1.shape[1]
    return pl.pallas_call(
        _mlp1_body,
        grid=(NBLK,),
        in_specs=[
            _fix_spec((1, 1)),
            _row_spec((RT, in_c)),
            _row_spec((RT, in_c)),
            _fix_spec((in_c, ch)),
            _fix_spec((1, ch)),
        ],
        out_specs=_row_spec((RT, ch)),
        out_shape=jax.ShapeDtypeStruct((N, ch), jnp.float32),
    )(eps, x, agg, w1, b1)


def _mlp2_call(a, w2, b2):
    ch = a.shape[1]
    co = w2.shape[1]
    return pl.pallas_call(
        _mlp2_body,
        grid=(NBLK,),
        in_specs=[
            _row_spec((RT, ch)),
            _fix_spec((ch, co)),
            _fix_spec((1, co)),
        ],
        out_specs=_row_spec((RT, co)),
        out_shape=jax.ShapeDtypeStruct((N, co), jnp.float32),
    )(a, w2, b2)


def _pool_call(batch3, x):
    ch = x.shape[1]
    return pl.pallas_call(
        _pool_body,
        grid=(NBLK,),
        in_specs=[
            pl.BlockSpec((1, 1, RT), lambda i: (i, 0, 0)),
            _row_spec((RT, ch)),
        ],
        out_specs=_fix_spec((G, ch)),
        out_shape=jax.ShapeDtypeStruct((G, ch), jnp.float32),
    )(batch3, x)


def _head_call(pooled, w1, b1, g, be, w2, b2):
    ch = pooled.shape[1]
    co = w2.shape[1]
    return pl.pallas_call(
        _head_body,
        grid=(1,),
        in_specs=[
            _fix_spec((G, ch)),
            _fix_spec((ch, ch)),
            _fix_spec((1, ch)),
            _fix_spec((1, ch)),
            _fix_spec((1, ch)),
            _fix_spec((ch, co)),
            _fix_spec((1, co)),
        ],
        out_specs=_fix_spec((G, co)),
        out_shape=jax.ShapeDtypeStruct((G, co), jnp.float32),
    )(pooled, w1, b1, g, be, w2, b2)


def _bn_ref(x, gamma, beta, eps=1e-5):
    m = jnp.mean(x, axis=0)
    v = jnp.var(x, axis=0)
    return (x - m) / jnp.sqrt(v + eps) * gamma + beta


def _forward(x, edge_index, batch, params):
    sel, counts = _compact_call(edge_index)
    for lp in params["layers"]:
        agg = _segment_sum(x, sel, counts)
        eps = lp["eps"].reshape(1, 1)
        h1 = _mlp1_call(eps, x, agg, lp["W1"], lp["b1"].reshape(1, -1))
        a = jax.nn.relu(_bn_ref(h1, lp["g1"], lp["be1"]))
        h2 = _mlp2_call(a, lp["W2"], lp["b2"].reshape(1, -1))
        x = jax.nn.relu(_bn_ref(h2, lp["g"], lp["be"]))
    batch3 = batch.reshape(NBLK, 1, RT)
    pooled = _pool_call(batch3, x)
    return _head_call(pooled, params["lin1_W"],
                      params["lin1_b"].reshape(1, -1),
                      params["bn1_g"].reshape(1, -1),
                      params["bn1_b"].reshape(1, -1),
                      params["lin2_W"], params["lin2_b"].reshape(1, -1))


def kernel(s_x, q_x, params, s_edge_index, q_edge_index, s_batch, q_batch,
           s_label):
    s_logits = _forward(s_x, s_edge_index, s_batch, params)
    q_logits = _forward(q_x, q_edge_index, q_batch, params)
    return (s_logits, q_logits, s_edge_index, s_x)


# speedup vs baseline: 1.4215x; 1.0035x over previous
import functools

import jax
import jax.numpy as jnp
from jax import lax
from jax.experimental import pallas as pl
from jax.experimental.pallas import tpu as pltpu
from jax.experimental.pallas import tpu_sc as plsc

N = 10000
E = 160000
G = 64

SC_NC = 2
SC_NS = 16
NW = SC_NC * SC_NS
NPASS = 2
NB = NPASS * NW
BR = 160
RB = 64
SELCAP_ROWS = 2560
CHUNK = 3200
NCHUNKS = E // CHUNK
STG_ROWS = 56
FLUSH_ROWS = 48
FLUSH_N = FLUSH_ROWS * RB

RT = 400
NBLK = N // RT


@functools.cache
def _sc_mesh():
    return plsc.VectorSubcoreMesh(core_axis_name="c", subcore_axis_name="s",
                                  num_cores=SC_NC, num_subcores=SC_NS)


def _compact_body(src_hbm, dst_hbm, sel_hbm, counts_hbm,
                  src_v, dst_v, stg0_v, stg1_v, cnt_v):
    c = lax.axis_index("c")
    t = lax.axis_index("s")
    w = c * SC_NS + t
    b0 = w
    b1 = NW + w
    lo0 = b0 * BR
    lo1 = b1 * BR

    def flush(stg_v, bkt, cnt, wr):
        def do(args):
            cnt_, wr_ = args
            wr_a = pl.multiple_of(wr_, 8)
            pltpu.sync_copy(stg_v.at[pl.ds(0, FLUSH_ROWS)],
                            sel_hbm.at[bkt, pl.ds(wr_a, FLUSH_ROWS)])
            for k in range(RB // 16):
                stg_v[0, pl.ds(k * 16, 16)] = stg_v[FLUSH_ROWS,
                                                    pl.ds(k * 16, 16)]
            return cnt_ - FLUSH_N, wr_ + FLUSH_ROWS
        return lax.cond(cnt >= FLUSH_N, do, lambda a: a, (cnt, wr))

    def chunk_loop(ch, carry):
        cnt0, wr0, cnt1, wr1 = carry
        pltpu.sync_copy(src_hbm.at[pl.ds(ch * CHUNK, CHUNK)], src_v)
        pltpu.sync_copy(dst_hbm.at[pl.ds(ch * CHUNK, CHUNK)], dst_v)

        def step(i, carry):
            cnt0, wr0, cnt1, wr1 = carry
            s = src_v[pl.ds(i * 16, 16)]
            d = dst_v[pl.ds(i * 16, 16)]

            def bucket(stg_v, lo, cnt):
                m = (d >= lo) & (d < lo + BR)
                mi = m.astype(jnp.int32)
                val = lax.shift_left(s, 8) | (d - lo)
                offs = cnt + plsc.cumsum(mi) - mi
                plsc.store_scatter(stg_v, [offs // RB, offs % RB], val,
                                   mask=m)
                return cnt + jnp.sum(mi)

            cnt0 = bucket(stg0_v, lo0, cnt0)
            cnt1 = bucket(stg1_v, lo1, cnt1)
            cnt0, wr0 = flush(stg0_v, b0, cnt0, wr0)
            cnt1, wr1 = flush(stg1_v, b1, cnt1, wr1)
            return cnt0, wr0, cnt1, wr1

        return lax.fori_loop(0, CHUNK // 16, step,
                             (cnt0, wr0, cnt1, wr1))

    z = jnp.int32(0)
    cnt0, wr0, cnt1, wr1 = lax.fori_loop(0, NCHUNKS, chunk_loop,
                                         (z, z, z, z))

    iota = lax.iota(jnp.int32, 16)
    trash = jnp.full((16,), BR, jnp.int32)

    def finish(stg_v, bkt, cnt, wr):
        for k in range(RB // 16):
            offs = cnt + k * 16 + iota
            plsc.store_scatter(stg_v, [offs // RB, offs % RB], trash)
        pltpu.sync_copy(stg_v,
                        sel_hbm.at[bkt, pl.ds(pl.multiple_of(wr, 8),
                                              STG_ROWS)])
        total = wr * RB + cnt
        cnt_v[...] = jnp.full((16,), total, jnp.int32)
        pltpu.sync_copy(cnt_v, counts_hbm.at[bkt])

    finish(stg0_v, b0, cnt0, wr0)
    finish(stg1_v, b1, cnt1, wr1)


@functools.cache
def _compact_kernel():
    return pl.kernel(
        _compact_body,
        out_type=(
            jax.ShapeDtypeStruct((NB, SELCAP_ROWS, RB), jnp.int32),
            jax.ShapeDtypeStruct((NB, 16), jnp.int32),
        ),
        mesh=_sc_mesh(),
        scratch_types=[
            pltpu.VMEM((CHUNK,), jnp.int32),
            pltpu.VMEM((CHUNK,), jnp.int32),
            pltpu.VMEM((STG_ROWS, RB), jnp.int32),
            pltpu.VMEM((STG_ROWS, RB), jnp.int32),
            pltpu.VMEM((16,), jnp.int32),
        ],
        compiler_params=pltpu.CompilerParams(needs_layout_passes=False),
    )


def _compact_call(edge_index):
    return _compact_kernel()(edge_index[0], edge_index[1])


def _agg_body(Hf, x_hbm, zeros_hbm, sel_hbm, counts_hbm, agg_hbm,
              sel_v, idx_v, ld_v, cnt_v, rows_v, acc_v, sem):
    c = lax.axis_index("c")
    t = lax.axis_index("s")
    w = c * SC_NS + t
    colv = lax.iota(jnp.int32, 16)
    vpr = Hf // 16

    for p in range(NPASS):
        b = p * NW + w

        pltpu.sync_copy(zeros_hbm, acc_v.at[pl.ds(0, BR)])

        pltpu.sync_copy(counts_hbm.at[b], cnt_v)
        m_tot = cnt_v[...][0]
        nblk = (m_tot + RB - 1) // RB
        nsb = (nblk + 15) // 16

        def sb_loop(sb, _):
            pltpu.sync_copy(
                sel_hbm.at[b, pl.ds(pl.multiple_of(sb * 16, 16), 16)], sel_v)
            hi = jnp.minimum(nblk - sb * 16, 16)

            def blk_loop(r, _):
                for k in range(RB // 16):
                    v = sel_v[r, pl.ds(k * 16, 16)]
                    idx_v[pl.ds(k * 16, 16)] = lax.shift_right_logical(v, 8)
                    ld_v[pl.ds(k * 16, 16)] = v & 255
                pltpu.async_copy(x_hbm.at[idx_v], rows_v, sem).wait()
                for g in range(RB // 16):
                    ldv = ld_v[pl.ds(g * 16, 16)]

                    def edge_loop(l4, _):
                        for dl in range(4):
                            l = l4 * 4 + dl
                            spl = jnp.take(ldv, jnp.full((16,), l, jnp.int32))
                            for j in range(vpr):
                                val = rows_v[g * 16 + l, pl.ds(j * 16, 16)]
                                plsc.addupdate_scatter(
                                    acc_v, [spl, j * 16 + colv], val)
                        return 0
                    lax.fori_loop(0, 4, edge_loop, 0)
                return 0
            lax.fori_loop(0, hi, blk_loop, 0)
            return 0
        lax.fori_loop(0, nsb, sb_loop, 0)

        base = pl.multiple_of(b * BR, 32)
        if p == 0:
            pltpu.sync_copy(acc_v.at[pl.ds(0, BR)],
                            agg_hbm.at[pl.ds(base, BR)])
        else:
            @pl.when(w < NW - 2)
            def _():
                pltpu.sync_copy(acc_v.at[pl.ds(0, BR)],
                                agg_hbm.at[pl.ds(base, BR)])

            @pl.when(w == NW - 2)
            def _():
                pltpu.sync_copy(acc_v.at[pl.ds(0, 80)],
                                agg_hbm.at[pl.ds(base, 80)])


@functools.cache
def _make_agg_call(Hf):
    return pl.kernel(
        functools.partial(_agg_body, Hf),
        out_type=jax.ShapeDtypeStruct((N, Hf), jnp.float32),
        mesh=_sc_mesh(),
        scratch_types=[
            pltpu.VMEM((16, RB), jnp.int32),
            pltpu.VMEM((RB,), jnp.int32),
            pltpu.VMEM((RB,), jnp.int32),
            pltpu.VMEM((16,), jnp.int32),
            pltpu.VMEM((RB, Hf), jnp.float32),
            pltpu.VMEM((BR + 8, Hf), jnp.float32),
            pltpu.SemaphoreType.DMA,
        ],
        compiler_params=pltpu.CompilerParams(needs_layout_passes=False),
    )


def _segment_sum(x, sel, counts):
    zeros = jnp.zeros((BR, x.shape[1]), jnp.float32)
    return _make_agg_call(x.shape[1])(x, zeros, sel, counts)


def _mlp1_body(eps_ref, x_ref, agg_ref, w1_ref, b1_ref, h1_ref):
    h_in = (1.0 + eps_ref[0, 0]) * x_ref[...] + agg_ref[...]
    h1 = jnp.dot(h_in, w1_ref[...], preferred_element_type=jnp.float32)
    h1_ref[...] = h1 + b1_ref[...]


def _mlp2_body(a_ref, w2_ref, b2_ref, h2_ref):
    h2 = jnp.dot(a_ref[...], w2_ref[...], preferred_element_type=jnp.float32)
    h2_ref[...] = h2 + b2_ref[...]


def _pool_body(b3_ref, x_ref, out_ref):
    i = pl.program_id(0)

    @pl.when(i == 0)
    def _():
        out_ref[...] = jnp.zeros_like(out_ref)

    gids = lax.broadcasted_iota(jnp.int32, (G, 1), 0)
    onehot = (b3_ref[0] == gids).astype(jnp.float32)
    out_ref[...] += jnp.dot(onehot, x_ref[...],
                            preferred_element_type=jnp.float32,
                            precision=lax.Precision.HIGHEST)


def _head_body(p_ref, w1_ref, b1_ref, g_ref, be_ref, w2_ref, b2_ref, out_ref):
    h = jnp.dot(p_ref[...], w1_ref[...], preferred_element_type=jnp.float32)
    h = h + b1_ref[...]
    mean = jnp.mean(h, axis=0, keepdims=True)
    var = jnp.mean(h * h, axis=0, keepdims=True) - mean * mean
    h = (h - mean) * lax.rsqrt(var + 1e-5) * g_ref[...] + be_ref[...]
    h = jnp.maximum(h, 0.0)
    out_ref[...] = jnp.dot(h, w2_ref[...],
                           preferred_element_type=jnp.float32) + b2_ref[...]


def _row_spec(bs):
    return pl.BlockSpec(bs, lambda i: (i, 0))


def _fix_spec(bs):
    return pl.BlockSpec(bs, lambda i: (0, 0))


def _mlp1_call(eps, x, agg, w1, b1):
    in_c = x.shape[1]
    ch = w1.shape[1]
    return pl.pallas_call(
        _mlp1_body,
        grid=(NBLK,),
        in_specs=[
            _fix_spec((1, 1)),
            _row_spec((RT, in_c)),
            _row_spec((RT, in_c)),
            _fix_spec((in_c, ch)),
            _fix_spec((1, ch)),
        ],
        out_specs=_row_spec((RT, ch)),
        out_shape=jax.ShapeDtypeStruct((N, ch), jnp.float32),
    )(eps, x, agg, w1, b1)


def _mlp2_call(a, w2, b2):
    ch = a.shape[1]
    co = w2.shape[1]
    return pl.pallas_call(
        _mlp2_body,
        grid=(NBLK,),
        in_specs=[
            _row_spec((RT, ch)),
            _fix_spec((ch, co)),
            _fix_spec((1, co)),
        ],
        out_specs=_row_spec((RT, co)),
        out_shape=jax.ShapeDtypeStruct((N, co), jnp.float32),
    )(a, w2, b2)


def _pool_call(batch3, x):
    ch = x.shape[1]
    return pl.pallas_call(
        _pool_body,
        grid=(NBLK,),
        in_specs=[
            pl.BlockSpec((1, 1, RT), lambda i: (i, 0, 0)),
            _row_spec((RT, ch)),
        ],
        out_specs=_fix_spec((G, ch)),
        out_shape=jax.ShapeDtypeStruct((G, ch), jnp.float32),
    )(batch3, x)


def _head_call(pooled, w1, b1, g, be, w2, b2):
    ch = pooled.shape[1]
    co = w2.shape[1]
    return pl.pallas_call(
        _head_body,
        grid=(1,),
        in_specs=[
            _fix_spec((G, ch)),
            _fix_spec((ch, ch)),
            _fix_spec((1, ch)),
            _fix_spec((1, ch)),
            _fix_spec((1, ch)),
            _fix_spec((ch, co)),
            _fix_spec((1, co)),
        ],
        out_specs=_fix_spec((G, co)),
        out_shape=jax.ShapeDtypeStruct((G, co), jnp.float32),
    )(pooled, w1, b1, g, be, w2, b2)


def _bn_ref(x, gamma, beta, eps=1e-5):
    m = jnp.mean(x, axis=0)
    v = jnp.var(x, axis=0)
    return (x - m) / jnp.sqrt(v + eps) * gamma + beta


def _forward(x, edge_index, batch, params):
    sel, counts = _compact_call(edge_index)
    for lp in params["layers"]:
        agg = _segment_sum(x, sel, counts)
        eps = lp["eps"].reshape(1, 1)
        h1 = _mlp1_call(eps, x, agg, lp["W1"], lp["b1"].reshape(1, -1))
        a = jax.nn.relu(_bn_ref(h1, lp["g1"], lp["be1"]))
        h2 = _mlp2_call(a, lp["W2"], lp["b2"].reshape(1, -1))
        x = jax.nn.relu(_bn_ref(h2, lp["g"], lp["be"]))
    batch3 = batch.reshape(NBLK, 1, RT)
    pooled = _pool_call(batch3, x)
    return _head_call(pooled, params["lin1_W"],
                      params["lin1_b"].reshape(1, -1),
                      params["bn1_g"].reshape(1, -1),
                      params["bn1_b"].reshape(1, -1),
                      params["lin2_W"], params["lin2_b"].reshape(1, -1))


def kernel(s_x, q_x, params, s_edge_index, q_edge_index, s_batch, q_batch,
           s_label):
    s_logits = _forward(s_x, s_edge_index, s_batch, params)
    q_logits = _forward(q_x, q_edge_index, q_batch, params)
    return (s_logits, q_logits, s_edge_index, s_x)
